# baseline (speedup 1.0000x reference)
import numpy as np
import ml_dtypes
import jax
import jax.numpy as jnp
from jax.experimental import pallas as pl
from jax.experimental.pallas import tpu as pltpu

_NUM_ANGLE = 100
_NUM_RHO = 100
_H = 100
_W = 100
_P = _H * _W
_P_CHUNKS = 5
_P_BLK = _P // _P_CHUNKS
_A_BLK = 8
_A_STEPS = 13
_K_PAD = _A_STEPS * _A_BLK * _NUM_RHO


def _onehot_t() -> np.ndarray:
    irho = float(int(np.sqrt(_H * _H + _W * _W) + 1)) / float(_NUM_RHO - 1)
    itheta = np.pi / _NUM_ANGLE
    angles = np.arange(_NUM_ANGLE, dtype=np.float64) * itheta
    tab_cos = (np.cos(angles) / irho).astype(np.float32)
    tab_sin = (np.sin(angles) / irho).astype(np.float32)
    ys, xs = np.meshgrid(np.arange(_H), np.arange(_W), indexing="ij")
    xx = (xs - (_W // 2)).reshape(-1).astype(np.float32)
    yy = (ys - (_H // 2)).reshape(-1).astype(np.float32)
    proj = xx[None, :] * tab_cos[:, None] + yy[None, :] * tab_sin[:, None]
    r = np.where(proj >= 0,
                 np.floor(proj + np.float32(0.5)),
                 np.ceil(proj - np.float32(0.5))).astype(np.int32) + _NUM_RHO // 2
    r = np.clip(r, 0, _NUM_RHO - 1)
    rk = r + (np.arange(_NUM_ANGLE, dtype=np.int32) * _NUM_RHO)[:, None]
    oht = np.zeros((_K_PAD, _P), dtype=ml_dtypes.bfloat16)
    cols = np.broadcast_to(np.arange(_P, dtype=np.int32), rk.shape)
    oht[rk.ravel(), cols.ravel()] = 1
    return np.ascontiguousarray(
        oht.reshape(_K_PAD, _P_CHUNKS, _P_BLK).transpose(1, 0, 2))


_OHT = _onehot_t()


def _hough_body(oht_ref, f_ref, o_ref):
    j = pl.program_id(1)
    f = f_ref[j]
    oht = oht_ref[0]
    acc = jax.lax.dot_general(
        f, oht, (((1,), (1,)), ((), ())),
        preferred_element_type=jnp.float32)
    acc = acc.reshape(o_ref.shape)

    @pl.when(j == 0)
    def _():
        o_ref[...] = acc

    @pl.when(j > 0)
    def _():
        o_ref[...] += acc


def kernel(feat):
    n, c, h, w = feat.shape
    nc = n * c
    f5 = (feat.reshape(nc, _P_CHUNKS, _P_BLK)
          .astype(jnp.bfloat16)
          .transpose(1, 0, 2))
    oht = jnp.asarray(_OHT)

    out = pl.pallas_call(
        _hough_body,
        grid=(_A_STEPS, _P_CHUNKS),
        in_specs=[
            pl.BlockSpec((1, _A_BLK * _NUM_RHO, _P_BLK), lambda i, j: (j, i, 0)),
            pl.BlockSpec((_P_CHUNKS, nc, _P_BLK), lambda i, j: (0, 0, 0)),
        ],
        out_specs=pl.BlockSpec((nc, _A_BLK, _NUM_RHO), lambda i, j: (0, i, 0)),
        out_shape=jax.ShapeDtypeStruct((nc, _NUM_ANGLE, _NUM_RHO), jnp.float32),
        compiler_params=pltpu.CompilerParams(
            dimension_semantics=("arbitrary", "arbitrary"),
        ),
    )(oht, f5)

    return out.reshape(n, c, _NUM_ANGLE, _NUM_RHO)


# speedup vs baseline: 1.1430x; 1.1430x over previous
import numpy as np
import ml_dtypes
import jax
import jax.numpy as jnp
from jax.experimental import pallas as pl
from jax.experimental.pallas import tpu as pltpu

_NUM_ANGLE = 100
_NUM_RHO = 100
_H = 100
_W = 100
_P = _H * _W
_A_BLK = 8
_A_STEPS = 13
_K_SPLIT = 2
_K_SUB = _A_BLK * _NUM_RHO // _K_SPLIT
_A_SUB = _A_BLK // _K_SPLIT


def _onehot_t() -> np.ndarray:
    irho = float(int(np.sqrt(_H * _H + _W * _W) + 1)) / float(_NUM_RHO - 1)
    itheta = np.pi / _NUM_ANGLE
    angles = np.arange(_NUM_ANGLE, dtype=np.float64) * itheta
    tab_cos = (np.cos(angles) / irho).astype(np.float32)
    tab_sin = (np.sin(angles) / irho).astype(np.float32)
    ys, xs = np.meshgrid(np.arange(_H), np.arange(_W), indexing="ij")
    xx = (xs - (_W // 2)).reshape(-1).astype(np.float32)
    yy = (ys - (_H // 2)).reshape(-1).astype(np.float32)
    proj = xx[None, :] * tab_cos[:, None] + yy[None, :] * tab_sin[:, None]
    r = np.where(proj >= 0,
                 np.floor(proj + np.float32(0.5)),
                 np.ceil(proj - np.float32(0.5))).astype(np.int32) + _NUM_RHO // 2
    r = np.clip(r, 0, _NUM_RHO - 1)
    rk = r + (np.arange(_NUM_ANGLE, dtype=np.int32) * _NUM_RHO)[:, None]
    k_pad = _A_STEPS * _A_BLK * _NUM_RHO
    oht = np.zeros((k_pad, _P), dtype=ml_dtypes.bfloat16)
    cols = np.broadcast_to(np.arange(_P, dtype=np.int32), rk.shape)
    oht[rk.ravel(), cols.ravel()] = 1
    return oht.reshape(k_pad // _K_SUB, _K_SUB, _P)


_OHT = _onehot_t()


def _hough_body(oht_ref, f_ref, o_ref):
    m = pl.program_id(1)
    f = f_ref[...]
    oht = oht_ref[0]
    acc = jax.lax.dot_general(
        f, oht, (((1,), (1,)), ((), ())),
        preferred_element_type=jnp.float32)
    acc = acc.reshape(acc.shape[0], _A_SUB, _NUM_RHO)

    @pl.when(m == 0)
    def _():
        o_ref[:, :_A_SUB, :] = acc

    @pl.when(m == 1)
    def _():
        o_ref[:, _A_SUB:, :] = acc


def kernel(feat):
    n, c, h, w = feat.shape
    nc = n * c
    feat2d = feat.reshape(nc, _P).astype(jnp.bfloat16)
    oht = jnp.asarray(_OHT)

    out = pl.pallas_call(
        _hough_body,
        grid=(_A_STEPS, _K_SPLIT),
        in_specs=[
            pl.BlockSpec((1, _K_SUB, _P), lambda i, m: (i * _K_SPLIT + m, 0, 0)),
            pl.BlockSpec((nc, _P), lambda i, m: (0, 0)),
        ],
        out_specs=pl.BlockSpec((nc, _A_BLK, _NUM_RHO), lambda i, m: (0, i, 0)),
        out_shape=jax.ShapeDtypeStruct((nc, _NUM_ANGLE, _NUM_RHO), jnp.float32),
        compiler_params=pltpu.CompilerParams(
            dimension_semantics=("arbitrary", "arbitrary"),
        ),
    )(oht, feat2d)

    return out.reshape(n, c, _NUM_ANGLE, _NUM_RHO)
